# baseline (speedup 1.0000x reference)
import functools

import jax
import jax.numpy as jnp
from jax import lax
from jax.experimental import pallas as pl
from jax.experimental.pallas import tpu as pltpu
from jax.experimental.pallas import tpu_sc as plsc

K_SPARSE = 64
EPS = 1e-8
L = 16


def _logits_body(h_ref, w2_ref, b2_ref, mem_ref, out_ref, st_out_ref, kn_ref):
    b = pl.program_id(0)

    @pl.when(b == 0)
    def _():
        h = h_ref[...]
        w2 = w2_ref[...]
        kf = lax.dot_general(h, w2, (((1,), (1,)), ((), ())),
                             preferred_element_type=jnp.float32)
        kf = kf + b2_ref[...][None, :]
        key = kf[:, :-1]
        st = kf[:, -1:]
        st = jnp.maximum(st, 0.0) + jnp.log1p(jnp.exp(-jnp.abs(st)))
        norm = jnp.sqrt(jnp.sum(key * key, axis=1, keepdims=True))
        kn_ref[...] = key / (norm + EPS)
        st_out_ref[...] = jnp.broadcast_to(st, st_out_ref.shape)

    mem = mem_ref[0]
    ns = jnp.sum(mem * mem, axis=1, keepdims=True)
    mn = mem / (jnp.sqrt(ns) + EPS)
    knb = kn_ref[pl.ds(b, 1), :]
    sim = lax.dot_general(knb, mn, (((1,), (1,)), ((), ())),
                          preferred_element_type=jnp.float32)
    out_ref[0] = sim


def _logits_call(h, memory, Wk, bk, Ws, bs, *, interpret=False):
    B, H = h.shape
    _, N, V = memory.shape
    W2 = jnp.concatenate([Wk, Ws], axis=0)
    b2 = jnp.concatenate([bk, bs], axis=0)
    return pl.pallas_call(
        _logits_body,
        grid=(B,),
        in_specs=[
            pl.BlockSpec((B, H), lambda b: (0, 0)),
            pl.BlockSpec((V + 1, H), lambda b: (0, 0)),
            pl.BlockSpec((V + 1,), lambda b: (0,)),
            pl.BlockSpec((1, N, V), lambda b: (b, 0, 0)),
        ],
        out_specs=[pl.BlockSpec((1, 1, N), lambda b: (b, 0, 0)),
                   pl.BlockSpec((B, 128), lambda b: (0, 0))],
        out_shape=[jax.ShapeDtypeStruct((B, 1, N), jnp.float32),
                   jax.ShapeDtypeStruct((B, 128), jnp.float32)],
        scratch_shapes=[pltpu.VMEM((B, V), jnp.float32)],
        interpret=interpret,
    )(h, W2, b2, memory)


def _splat_i32(x):
    return lax.broadcast_in_dim(jnp.int32(x) if isinstance(x, int) else x, (L,), ())


def _splat_u32(x):
    return lax.broadcast_in_dim(x, (L,), ())


def _splat_f32(x):
    return lax.broadcast_in_dim(x, (L,), ())


def _sc_body(lg_hbm, st_hbm, memflat_hbm, w_hbm, rv_hbm,
             lg_ref, st_ref, u_ref, cval_ref, cidx_ref, row_ref, sm_ref,
             idx_ref, rows_ref, acc_ref, sem):
    N = 1024
    NV = N // L
    b = lax.axis_index("s") * 2 + lax.axis_index("c")
    base = b * N

    pltpu.sync_copy(lg_hbm.at[pl.ds(base, N)], lg_ref)
    pltpu.sync_copy(st_hbm, st_ref)
    stv = plsc.load_gather(st_ref, [lax.broadcast_in_dim(b, (L,), ())])

    for j in range(NV):
        x = lg_ref[pl.ds(j * L, L)] * stv
        lg_ref[pl.ds(j * L, L)] = x
        s = lax.bitcast_convert_type(x, jnp.int32)
        m = lax.shift_right_arithmetic(s, _splat_i32(31))
        u = s ^ (m | _splat_i32(-(2 ** 31)))
        u_ref[pl.ds(j * L, L)] = lax.bitcast_convert_type(u, jnp.uint32)

    kk = jnp.int32(K_SPARSE)

    def bit_body(i, t):
        bit = lax.convert_element_type(31 - i, jnp.uint32)
        tc = t | lax.shift_left(jnp.uint32(1), bit)
        tcv = _splat_u32(tc)
        onev = _splat_i32(1)
        zerov = _splat_i32(0)
        acc = zerov
        for j in range(NV):
            uv = u_ref[pl.ds(j * L, L)]
            acc = acc + jnp.where(uv >= tcv, onev, zerov)
        c = jnp.sum(acc)
        return jnp.where(c >= kk, tc, t)

    t = lax.fori_loop(0, 32, bit_body, jnp.uint32(0), unroll=False)
    tv = _splat_u32(t)

    iota = lax.broadcasted_iota(jnp.int32, (L,), 0)
    onev = _splat_i32(1)
    zerov = _splat_i32(0)
    offv = zerov
    for phase in range(2):
        for j in range(NV):
            uv = u_ref[pl.ds(j * L, L)]
            mask = (uv > tv) if phase == 0 else (uv == tv)
            mi = jnp.where(mask, onev, zerov)
            pos = offv + plsc.cumsum(mi) - onev
            plsc.store_scatter(cval_ref, [pos], lg_ref[pl.ds(j * L, L)], mask=mask)
            plsc.store_scatter(cidx_ref, [pos], iota + _splat_i32(j * L), mask=mask)
            offv = offv + plsc.all_reduce_population_count(mask)

    KV = K_SPARSE // L
    vals = [cval_ref[pl.ds(j * L, L)] for j in range(KV)]
    m = vals[0]
    for v in vals[1:]:
        m = jnp.maximum(m, v)
    mx = jnp.max(m)
    mxv = _splat_f32(mx)
    exps = [jnp.exp(v - mxv) for v in vals]
    ssum = exps[0]
    for e in exps[1:]:
        ssum = ssum + e
    sv = _splat_f32(jnp.sum(ssum))
    invv = lax.broadcast_in_dim(jnp.float32(1.0), (L,), ()) / sv
    sms = [e * invv for e in exps]
    for j in range(KV):
        sm_ref[pl.ds(j * L, L)] = sms[j]

    zf = _splat_f32(jnp.float32(0.0))
    for j in range(NV):
        row_ref[pl.ds(j * L, L)] = zf
    basev = _splat_i32(base)
    for j in range(KV):
        cj = cidx_ref[pl.ds(j * L, L)]
        plsc.store_scatter(row_ref, [cj], sms[j])
        idx_ref[pl.ds(j * L, L)] = cj + basev
    pltpu.sync_copy(row_ref, w_hbm.at[pl.ds(base, N)])

    pltpu.async_copy(memflat_hbm.at[idx_ref], rows_ref, sem).wait()

    V = 256
    VV = V // L

    def rbody(j, accs):
        wv = plsc.load_gather(sm_ref, [lax.broadcast_in_dim(j, (L,), ())])
        return tuple(accs[v] + wv * rows_ref[j, pl.ds(v * L, L)]
                     for v in range(VV))

    accs = lax.fori_loop(0, K_SPARSE, rbody,
                         tuple(zf for _ in range(VV)), unroll=False)
    for v in range(VV):
        acc_ref[pl.ds(v * L, L)] = accs[v]
    pltpu.sync_copy(acc_ref, rv_hbm.at[pl.ds(b * V, V)])


def _sc_call(sim_flat, st, memflat):
    N = 1024
    V = memflat.shape[1]
    B = sim_flat.shape[0] // N
    mesh = plsc.VectorSubcoreMesh(core_axis_name="c", subcore_axis_name="s",
                                  num_cores=2, num_subcores=16)
    f = functools.partial(
        pl.kernel,
        out_type=[jax.ShapeDtypeStruct((B * N,), jnp.float32),
                  jax.ShapeDtypeStruct((B * V,), jnp.float32)],
        mesh=mesh,
        compiler_params=pltpu.CompilerParams(needs_layout_passes=False),
        scratch_types=[
            pltpu.VMEM((N,), jnp.float32),
            pltpu.VMEM((B,), jnp.float32),
            pltpu.VMEM((N,), jnp.uint32),
            pltpu.VMEM((N,), jnp.float32),
            pltpu.VMEM((N,), jnp.int32),
            pltpu.VMEM((N,), jnp.float32),
            pltpu.VMEM((K_SPARSE,), jnp.float32),
            pltpu.VMEM((K_SPARSE,), jnp.int32),
            pltpu.VMEM((K_SPARSE, V), jnp.float32),
            pltpu.VMEM((V,), jnp.float32),
            pltpu.SemaphoreType.DMA,
        ],
    )(_sc_body)
    return f(sim_flat, st, memflat)


def kernel(h, memory, prev_read_weights, link_matrix, Wk, bk, Ws, bs):
    B, N, V = memory.shape
    sim, stv = _logits_call(h, memory, Wk, bk, Ws, bs)
    w_flat, rv_flat = _sc_call(sim.reshape(B * N), stv[:, 0],
                               memory.reshape(B * N, V))
    return rv_flat.reshape(B, V), w_flat.reshape(B, N)
